# baseline (speedup 1.0000x reference)
import functools

import jax
import jax.numpy as jnp
from jax import lax
from jax.experimental import pallas as pl
from jax.experimental.pallas import tpu as pltpu
from jax.experimental.pallas import tpu_sc as plsc

_N = 8192
_N_NEW = 4096
_NH = 64
_D3 = 3 * _NH
_DIAG_B = 128
_ROW_B = 256

_f32 = jnp.float32


_SC_NW = 32
_SC_PER_W = _N // _SC_NW
_SC_CHUNK = 128


def _sc_diag_body(node_ref, edge_ref, dn_ref, de_ref, idx_v, val_v, sem):
    wid = lax.axis_index("s") * 2 + lax.axis_index("c")
    base = wid * _SC_PER_W
    for c in range(_SC_PER_W // _SC_CHUNK):
        for j in range(_SC_CHUNK // 16):
            lanes = lax.iota(jnp.int32, 16)
            idx_v[pl.ds(j * 16, 16)] = (base + c * _SC_CHUNK + j * 16 + lanes) * (_N + 1)
        pltpu.async_copy(node_ref.at[idx_v], val_v, sem).wait()
        pltpu.sync_copy(val_v, dn_ref.at[pl.ds(base + c * _SC_CHUNK, _SC_CHUNK)])
        pltpu.async_copy(edge_ref.at[idx_v], val_v, sem).wait()
        pltpu.sync_copy(val_v, de_ref.at[pl.ds(base + c * _SC_CHUNK, _SC_CHUNK)])


def _sc_extract_diags(node_adj, edge_adj):
    k = functools.partial(
        pl.kernel,
        out_type=[jax.ShapeDtypeStruct((_N,), _f32),
                  jax.ShapeDtypeStruct((_N,), _f32)],
        mesh=plsc.VectorSubcoreMesh(core_axis_name="c", subcore_axis_name="s"),
        scratch_types=[pltpu.VMEM((_SC_CHUNK,), jnp.int32),
                       pltpu.VMEM((_SC_CHUNK,), _f32),
                       pltpu.SemaphoreType.DMA],
    )(_sc_diag_body)
    dn, de = k(node_adj.reshape(-1), edge_adj.reshape(-1))
    return dn.reshape(_N, 1), de.reshape(_N, 1)


def _diag_body(node_ref, edge_ref, dn_ref, de_ref):
    b = _DIAG_B
    rows = jax.lax.broadcasted_iota(jnp.int32, (b, b), 0)
    cols = jax.lax.broadcasted_iota(jnp.int32, (b, b), 1)
    eye = rows == cols
    dn_ref[:, :] = jnp.sum(jnp.where(eye, node_ref[:, :], 0.0), axis=1, keepdims=True)
    de_ref[:, :] = jnp.sum(jnp.where(eye, edge_ref[:, :], 0.0), axis=1, keepdims=True)


def _extract_diags(node_adj, edge_adj):
    nblk = _N // _DIAG_B
    return pl.pallas_call(
        _diag_body,
        grid=(nblk,),
        in_specs=[
            pl.BlockSpec((_DIAG_B, _DIAG_B), lambda i: (i, i)),
            pl.BlockSpec((_DIAG_B, _DIAG_B), lambda i: (i, i)),
        ],
        out_specs=[
            pl.BlockSpec((_DIAG_B, 1), lambda i: (i, 0)),
            pl.BlockSpec((_DIAG_B, 1), lambda i: (i, 0)),
        ],
        out_shape=[
            jax.ShapeDtypeStruct((_N, 1), _f32),
            jax.ShapeDtypeStruct((_N, 1), _f32),
        ],
        compiler_params=pltpu.CompilerParams(
            dimension_semantics=("parallel",)),
    )(node_adj, edge_adj)


def _it_body(x0_ref, x1_ref, x2_ref, *rest):
    param_refs = rest[:18]
    dtail_ref = rest[18]
    hin_ref = rest[19]
    out_ref = rest[20]
    xs = (x0_ref, x1_ref, x2_ref)
    out_ref[0:_N_NEW, :] = hin_ref[:, :]
    for i in range(3):
        w1t, b1, gamma, beta, w2t, b2 = param_refs[6 * i:6 * i + 6]
        h1 = jnp.dot(xs[i][:, :], w1t[:, :], preferred_element_type=_f32) + b1[:, :]
        mu = jnp.mean(h1, axis=0, keepdims=True)
        var = jnp.mean((h1 - mu) ** 2, axis=0, keepdims=True)
        hn = (h1 - mu) / jnp.sqrt(var + 1e-5) * gamma[:, :] + beta[:, :]
        hr = jnp.maximum(hn, 0.0)
        h2 = jnp.dot(hr, w2t[:, :], preferred_element_type=_f32) + b2[:, :]
        out_ref[_N_NEW:_N, _NH * i:_NH * (i + 1)] = dtail_ref[:, :] * h2


def _input_transform(x, it_params, d_tail, h_in):
    x0 = x[:, 0:8]
    x1 = jnp.pad(x[:, 8:10], ((0, 0), (0, 6)))
    x2 = x[:, 10:138]
    args = [x0, x1, x2]
    for i in range(3):
        p = it_params[i]
        w1 = p["W1"]
        if w1.shape[1] == 2:
            w1 = jnp.pad(w1, ((0, 0), (0, 6)))
        args.append(w1.T)
        args.append(p["b1"].reshape(1, _NH))
        args.append(p["gamma"].reshape(1, _NH))
        args.append(p["beta"].reshape(1, _NH))
        args.append(p["W2"].T)
        args.append(p["b2"].reshape(1, _NH))
    args.append(d_tail)
    args.append(h_in)
    return pl.pallas_call(
        _it_body,
        out_shape=jax.ShapeDtypeStruct((_N, _D3), _f32),
    )(*args)


def _mega_body(node_ref, edge_ref, hf_ref, dn_ref, de_ref,
               wzt_ref, uzt_ref, wrt_ref, urt_ref, wnt_ref, unt_ref,
               bz_ref, br_ref, bn_ref, wno_ref, weo_ref, bno_ref, beo_ref,
               ho_ref, z0_ref, z1_ref, z2_ref, y_ref, sig_ref):
    i = pl.program_id(0)
    row0 = i * _ROW_B
    a = node_ref[:, :] + edge_ref[:, :]
    m = jnp.dot(a, hf_ref[:, :], preferred_element_type=_f32)
    h = hf_ref[pl.ds(row0, _ROW_B), :]
    dn = dn_ref[pl.ds(row0, _ROW_B), :]
    de = de_ref[pl.ds(row0, _ROW_B), :]
    z = jax.nn.sigmoid(jnp.dot(m, wzt_ref[:, :], preferred_element_type=_f32)
                       + jnp.dot(h, uzt_ref[:, :], preferred_element_type=_f32)
                       + bz_ref[:, :])
    r = jax.nn.sigmoid(jnp.dot(m, wrt_ref[:, :], preferred_element_type=_f32)
                       + jnp.dot(h, urt_ref[:, :], preferred_element_type=_f32)
                       + br_ref[:, :])
    n = jnp.tanh(jnp.dot(m, wnt_ref[:, :], preferred_element_type=_f32)
                 + jnp.dot(r * h, unt_ref[:, :], preferred_element_type=_f32)
                 + bn_ref[:, :])
    ho = (1.0 - z) * h + z * n
    yv = (dn * (jnp.dot(ho, wno_ref[:, :], preferred_element_type=_f32)
                + bno_ref[:, :])
          + de * (jnp.dot(ho, weo_ref[:, :], preferred_element_type=_f32)
                  + beo_ref[:, :]))
    ho_ref[:, :] = ho
    z0_ref[:, :] = z[:, 0:_NH]
    z1_ref[:, :] = z[:, _NH:2 * _NH]
    z2_ref[:, :] = z[:, 2 * _NH:3 * _NH]
    y_ref[:, :] = yv
    sig_ref[:, :] = jax.nn.sigmoid(yv)


def _block_diag_t(mats):
    out = jnp.zeros((_D3, _D3), _f32)
    for i, m in enumerate(mats):
        out = out.at[_NH * i:_NH * (i + 1), _NH * i:_NH * (i + 1)].set(m.T)
    return out


def _mega(node_adj, edge_adj, h_full, dn, de, gru_params, out_node, out_edge):
    nblk = _N // _ROW_B
    wargs = []
    for name in ("Wz", "Uz", "Wr", "Ur", "Wn", "Un"):
        wargs.append(_block_diag_t([gru_params[i][name] for i in range(3)]))
    for name in ("bz", "br", "bn"):
        wargs.append(jnp.concatenate(
            [gru_params[i][name] for i in range(3)]).reshape(1, _D3))
    wargs.append(out_node["W"].T)
    wargs.append(out_edge["W"].T)
    wargs.append(out_node["b"].reshape(1, 1))
    wargs.append(out_edge["b"].reshape(1, 1))

    row_spec = pl.BlockSpec((_ROW_B, _N), lambda i: (i, 0))
    full_spec = lambda shape: pl.BlockSpec(shape, lambda i: (0, 0))
    in_specs = [
        row_spec,
        row_spec,
        full_spec((_N, _D3)),
        full_spec((_N, 1)),
        full_spec((_N, 1)),
    ]
    in_specs += [full_spec((_D3, _D3))] * 6
    in_specs += [full_spec((1, _D3))] * 3
    in_specs += [full_spec((_D3, 1))] * 2
    in_specs += [full_spec((1, 1))] * 2
    out_specs = [
        pl.BlockSpec((_ROW_B, _D3), lambda i: (i, 0)),
        pl.BlockSpec((_ROW_B, _NH), lambda i: (i, 0)),
        pl.BlockSpec((_ROW_B, _NH), lambda i: (i, 0)),
        pl.BlockSpec((_ROW_B, _NH), lambda i: (i, 0)),
        pl.BlockSpec((_ROW_B, 1), lambda i: (i, 0)),
        pl.BlockSpec((_ROW_B, 1), lambda i: (i, 0)),
    ]
    out_shape = [
        jax.ShapeDtypeStruct((_N, _D3), _f32),
        jax.ShapeDtypeStruct((_N, _NH), _f32),
        jax.ShapeDtypeStruct((_N, _NH), _f32),
        jax.ShapeDtypeStruct((_N, _NH), _f32),
        jax.ShapeDtypeStruct((_N, 1), _f32),
        jax.ShapeDtypeStruct((_N, 1), _f32),
    ]
    return pl.pallas_call(
        _mega_body,
        grid=(nblk,),
        in_specs=in_specs,
        out_specs=out_specs,
        out_shape=out_shape,
        compiler_params=pltpu.CompilerParams(
            dimension_semantics=("parallel",),
            vmem_limit_bytes=63 * 1024 * 1024),
    )(node_adj, edge_adj, h_full, dn, de, *wargs)


def kernel(x, h_in, node_adj, edge_adj, params):
    dn, de = _sc_extract_diags(node_adj, edge_adj)
    d_tail = dn[_N_NEW:]
    h_full = _input_transform(x, params["it"], d_tail, h_in)
    ho, z0, z1, z2, y, sig = _mega(node_adj, edge_adj, h_full, dn, de,
                                   params["gru"], params["out_node"],
                                   params["out_edge"])
    return sig, y, ho, (z0, z1, z2)


# speedup vs baseline: 2.0992x; 2.0992x over previous
import functools

import jax
import jax.numpy as jnp
from jax import lax
from jax.experimental import pallas as pl
from jax.experimental.pallas import tpu as pltpu
from jax.experimental.pallas import tpu_sc as plsc

_N = 8192
_N_NEW = 4096
_NH = 64
_D3 = 3 * _NH
_DIAG_B = 128
_ROW_B = 256

_f32 = jnp.float32


_SC_NW = 32
_SC_PER_W = _N // _SC_NW
_SC_CHUNK = 128


def _sc_diag_body(node_ref, edge_ref, dn_ref, de_ref, idx_v, val_v, sem):
    wid = lax.axis_index("s") * 2 + lax.axis_index("c")
    base = wid * _SC_PER_W
    for c in range(_SC_PER_W // _SC_CHUNK):
        for j in range(_SC_CHUNK // 16):
            lanes = lax.iota(jnp.int32, 16)
            idx_v[pl.ds(j * 16, 16)] = (base + c * _SC_CHUNK + j * 16 + lanes) * (_N + 1)
        pltpu.async_copy(node_ref.at[idx_v], val_v, sem).wait()
        pltpu.sync_copy(val_v, dn_ref.at[pl.ds(base + c * _SC_CHUNK, _SC_CHUNK)])
        pltpu.async_copy(edge_ref.at[idx_v], val_v, sem).wait()
        pltpu.sync_copy(val_v, de_ref.at[pl.ds(base + c * _SC_CHUNK, _SC_CHUNK)])


def _sc_extract_diags(node_adj, edge_adj):
    k = functools.partial(
        pl.kernel,
        out_type=[jax.ShapeDtypeStruct((_N,), _f32),
                  jax.ShapeDtypeStruct((_N,), _f32)],
        mesh=plsc.VectorSubcoreMesh(core_axis_name="c", subcore_axis_name="s"),
        scratch_types=[pltpu.VMEM((_SC_CHUNK,), jnp.int32),
                       pltpu.VMEM((_SC_CHUNK,), _f32),
                       pltpu.SemaphoreType.DMA],
    )(_sc_diag_body)
    dn, de = k(node_adj.reshape(-1), edge_adj.reshape(-1))
    return dn.reshape(_N, 1), de.reshape(_N, 1)


def _diag_body(node_ref, edge_ref, dn_ref, de_ref):
    b = _DIAG_B
    rows = jax.lax.broadcasted_iota(jnp.int32, (b, b), 0)
    cols = jax.lax.broadcasted_iota(jnp.int32, (b, b), 1)
    eye = rows == cols
    dn_ref[:, :] = jnp.sum(jnp.where(eye, node_ref[:, :], 0.0), axis=1, keepdims=True)
    de_ref[:, :] = jnp.sum(jnp.where(eye, edge_ref[:, :], 0.0), axis=1, keepdims=True)


def _extract_diags(node_adj, edge_adj):
    nblk = _N // _DIAG_B
    return pl.pallas_call(
        _diag_body,
        grid=(nblk,),
        in_specs=[
            pl.BlockSpec((_DIAG_B, _DIAG_B), lambda i: (i, i)),
            pl.BlockSpec((_DIAG_B, _DIAG_B), lambda i: (i, i)),
        ],
        out_specs=[
            pl.BlockSpec((_DIAG_B, 1), lambda i: (i, 0)),
            pl.BlockSpec((_DIAG_B, 1), lambda i: (i, 0)),
        ],
        out_shape=[
            jax.ShapeDtypeStruct((_N, 1), _f32),
            jax.ShapeDtypeStruct((_N, 1), _f32),
        ],
        compiler_params=pltpu.CompilerParams(
            dimension_semantics=("parallel",)),
    )(node_adj, edge_adj)


def _it_body(x0_ref, x1_ref, x2_ref, *rest):
    param_refs = rest[:18]
    dtail_ref = rest[18]
    hin_ref = rest[19]
    out_ref = rest[20]
    xs = (x0_ref, x1_ref, x2_ref)
    out_ref[0:_N_NEW, :] = hin_ref[:, :]
    for i in range(3):
        w1t, b1, gamma, beta, w2t, b2 = param_refs[6 * i:6 * i + 6]
        h1 = jnp.dot(xs[i][:, :], w1t[:, :], preferred_element_type=_f32) + b1[:, :]
        mu = jnp.mean(h1, axis=0, keepdims=True)
        var = jnp.mean((h1 - mu) ** 2, axis=0, keepdims=True)
        hn = (h1 - mu) / jnp.sqrt(var + 1e-5) * gamma[:, :] + beta[:, :]
        hr = jnp.maximum(hn, 0.0)
        h2 = jnp.dot(hr, w2t[:, :], preferred_element_type=_f32) + b2[:, :]
        out_ref[_N_NEW:_N, _NH * i:_NH * (i + 1)] = dtail_ref[:, :] * h2


def _input_transform(x, it_params, d_tail, h_in):
    x0 = x[:, 0:8]
    x1 = jnp.pad(x[:, 8:10], ((0, 0), (0, 6)))
    x2 = x[:, 10:138]
    args = [x0, x1, x2]
    for i in range(3):
        p = it_params[i]
        w1 = p["W1"]
        if w1.shape[1] == 2:
            w1 = jnp.pad(w1, ((0, 0), (0, 6)))
        args.append(w1.T)
        args.append(p["b1"].reshape(1, _NH))
        args.append(p["gamma"].reshape(1, _NH))
        args.append(p["beta"].reshape(1, _NH))
        args.append(p["W2"].T)
        args.append(p["b2"].reshape(1, _NH))
    args.append(d_tail)
    args.append(h_in)
    return pl.pallas_call(
        _it_body,
        out_shape=jax.ShapeDtypeStruct((_N, _D3), _f32),
    )(*args)


def _mega_body(n0_ref, n1_ref, e0_ref, e1_ref, hf_ref, dn_ref, de_ref,
               wzt_ref, uzt_ref, wrt_ref, urt_ref, wnt_ref, unt_ref,
               bz_ref, br_ref, bn_ref, wno_ref, weo_ref, bno_ref, beo_ref,
               ho_ref, z0_ref, z1_ref, z2_ref, y_ref, sig_ref):
    i = pl.program_id(0)
    row0 = i * _ROW_B
    a = jnp.concatenate([n0_ref[:, :] + e0_ref[:, :],
                         n1_ref[:, :] + e1_ref[:, :]], axis=0)
    m = jnp.dot(a, hf_ref[:, :], preferred_element_type=_f32)
    h = hf_ref[pl.ds(row0, _ROW_B), :]
    dn = dn_ref[pl.ds(row0, _ROW_B), :]
    de = de_ref[pl.ds(row0, _ROW_B), :]
    z = jax.nn.sigmoid(jnp.dot(m, wzt_ref[:, :], preferred_element_type=_f32)
                       + jnp.dot(h, uzt_ref[:, :], preferred_element_type=_f32)
                       + bz_ref[:, :])
    r = jax.nn.sigmoid(jnp.dot(m, wrt_ref[:, :], preferred_element_type=_f32)
                       + jnp.dot(h, urt_ref[:, :], preferred_element_type=_f32)
                       + br_ref[:, :])
    n = jnp.tanh(jnp.dot(m, wnt_ref[:, :], preferred_element_type=_f32)
                 + jnp.dot(r * h, unt_ref[:, :], preferred_element_type=_f32)
                 + bn_ref[:, :])
    ho = (1.0 - z) * h + z * n
    yv = (dn * (jnp.dot(ho, wno_ref[:, :], preferred_element_type=_f32)
                + bno_ref[:, :])
          + de * (jnp.dot(ho, weo_ref[:, :], preferred_element_type=_f32)
                  + beo_ref[:, :]))
    ho_ref[:, :] = ho
    z0_ref[:, :] = z[:, 0:_NH]
    z1_ref[:, :] = z[:, _NH:2 * _NH]
    z2_ref[:, :] = z[:, 2 * _NH:3 * _NH]
    y_ref[:, :] = yv
    sig_ref[:, :] = jax.nn.sigmoid(yv)


def _block_diag_t(mats):
    out = jnp.zeros((_D3, _D3), _f32)
    for i, m in enumerate(mats):
        out = out.at[_NH * i:_NH * (i + 1), _NH * i:_NH * (i + 1)].set(m.T)
    return out


def _mega(node_adj, edge_adj, h_full, dn, de, gru_params, out_node, out_edge):
    nblk = _N // _ROW_B
    wargs = []
    for name in ("Wz", "Uz", "Wr", "Ur", "Wn", "Un"):
        wargs.append(_block_diag_t([gru_params[i][name] for i in range(3)]))
    for name in ("bz", "br", "bn"):
        wargs.append(jnp.concatenate(
            [gru_params[i][name] for i in range(3)]).reshape(1, _D3))
    wargs.append(out_node["W"].T)
    wargs.append(out_edge["W"].T)
    wargs.append(out_node["b"].reshape(1, 1))
    wargs.append(out_edge["b"].reshape(1, 1))

    sub0 = pl.BlockSpec((_ROW_B // 2, _N), lambda i: (2 * i, 0))
    sub1 = pl.BlockSpec((_ROW_B // 2, _N), lambda i: (2 * i + 1, 0))
    full_spec = lambda shape: pl.BlockSpec(shape, lambda i: (0, 0))
    in_specs = [
        sub0,
        sub1,
        sub0,
        sub1,
        full_spec((_N, _D3)),
        full_spec((_N, 1)),
        full_spec((_N, 1)),
    ]
    in_specs += [full_spec((_D3, _D3))] * 6
    in_specs += [full_spec((1, _D3))] * 3
    in_specs += [full_spec((_D3, 1))] * 2
    in_specs += [full_spec((1, 1))] * 2
    out_specs = [
        pl.BlockSpec((_ROW_B, _D3), lambda i: (i, 0)),
        pl.BlockSpec((_ROW_B, _NH), lambda i: (i, 0)),
        pl.BlockSpec((_ROW_B, _NH), lambda i: (i, 0)),
        pl.BlockSpec((_ROW_B, _NH), lambda i: (i, 0)),
        pl.BlockSpec((_ROW_B, 1), lambda i: (i, 0)),
        pl.BlockSpec((_ROW_B, 1), lambda i: (i, 0)),
    ]
    out_shape = [
        jax.ShapeDtypeStruct((_N, _D3), _f32),
        jax.ShapeDtypeStruct((_N, _NH), _f32),
        jax.ShapeDtypeStruct((_N, _NH), _f32),
        jax.ShapeDtypeStruct((_N, _NH), _f32),
        jax.ShapeDtypeStruct((_N, 1), _f32),
        jax.ShapeDtypeStruct((_N, 1), _f32),
    ]
    return pl.pallas_call(
        _mega_body,
        grid=(nblk,),
        in_specs=in_specs,
        out_specs=out_specs,
        out_shape=out_shape,
        compiler_params=pltpu.CompilerParams(
            dimension_semantics=("parallel",),
            vmem_limit_bytes=63 * 1024 * 1024),
    )(node_adj, node_adj, edge_adj, edge_adj, h_full, dn, de, *wargs)


def kernel(x, h_in, node_adj, edge_adj, params):
    dn, de = _extract_diags(node_adj, edge_adj)
    d_tail = dn[_N_NEW:]
    h_full = _input_transform(x, params["it"], d_tail, h_in)
    ho, z0, z1, z2, y, sig = _mega(node_adj, edge_adj, h_full, dn, de,
                                   params["gru"], params["out_node"],
                                   params["out_edge"])
    return sig, y, ho, (z0, z1, z2)


# speedup vs baseline: 2.1161x; 1.0080x over previous
import functools

import jax
import jax.numpy as jnp
from jax import lax
from jax.experimental import pallas as pl
from jax.experimental.pallas import tpu as pltpu
from jax.experimental.pallas import tpu_sc as plsc

_N = 8192
_N_NEW = 4096
_NH = 64
_D3 = 3 * _NH
_DIAG_B = 128
_ROW_B = 256

_f32 = jnp.float32


_SC_NW = 32
_SC_PER_W = _N // _SC_NW
_SC_CHUNK = 128


def _sc_diag_body(node_ref, edge_ref, dn_ref, de_ref, idx_v, val_v, sem):
    wid = lax.axis_index("s") * 2 + lax.axis_index("c")
    base = wid * _SC_PER_W
    for c in range(_SC_PER_W // _SC_CHUNK):
        for j in range(_SC_CHUNK // 16):
            lanes = lax.iota(jnp.int32, 16)
            idx_v[pl.ds(j * 16, 16)] = (base + c * _SC_CHUNK + j * 16 + lanes) * (_N + 1)
        pltpu.async_copy(node_ref.at[idx_v], val_v, sem).wait()
        pltpu.sync_copy(val_v, dn_ref.at[pl.ds(base + c * _SC_CHUNK, _SC_CHUNK)])
        pltpu.async_copy(edge_ref.at[idx_v], val_v, sem).wait()
        pltpu.sync_copy(val_v, de_ref.at[pl.ds(base + c * _SC_CHUNK, _SC_CHUNK)])


def _sc_extract_diags(node_adj, edge_adj):
    k = functools.partial(
        pl.kernel,
        out_type=[jax.ShapeDtypeStruct((_N,), _f32),
                  jax.ShapeDtypeStruct((_N,), _f32)],
        mesh=plsc.VectorSubcoreMesh(core_axis_name="c", subcore_axis_name="s"),
        scratch_types=[pltpu.VMEM((_SC_CHUNK,), jnp.int32),
                       pltpu.VMEM((_SC_CHUNK,), _f32),
                       pltpu.SemaphoreType.DMA],
    )(_sc_diag_body)
    dn, de = k(node_adj.reshape(-1), edge_adj.reshape(-1))
    return dn.reshape(_N, 1), de.reshape(_N, 1)


def _diag_body(node_ref, edge_ref, dn_ref, de_ref, dt_ref):
    b = _DIAG_B
    rows = jax.lax.broadcasted_iota(jnp.int32, (b, b), 0)
    cols = jax.lax.broadcasted_iota(jnp.int32, (b, b), 1)
    eye = rows == cols
    d_node = jnp.sum(jnp.where(eye, node_ref[:, :], 0.0), axis=1, keepdims=True)
    dn_ref[:, :] = d_node
    de_ref[:, :] = jnp.sum(jnp.where(eye, edge_ref[:, :], 0.0), axis=1, keepdims=True)
    dt_ref[:, :] = d_node


def _extract_diags(node_adj, edge_adj):
    nblk = _N // _DIAG_B
    tail0 = _N_NEW // _DIAG_B
    return pl.pallas_call(
        _diag_body,
        grid=(nblk,),
        in_specs=[
            pl.BlockSpec((_DIAG_B, _DIAG_B), lambda i: (i, i)),
            pl.BlockSpec((_DIAG_B, _DIAG_B), lambda i: (i, i)),
        ],
        out_specs=[
            pl.BlockSpec((_DIAG_B, 1), lambda i: (i, 0)),
            pl.BlockSpec((_DIAG_B, 1), lambda i: (i, 0)),
            pl.BlockSpec((_DIAG_B, 1), lambda i: (jnp.maximum(i - tail0, 0), 0)),
        ],
        out_shape=[
            jax.ShapeDtypeStruct((_N, 1), _f32),
            jax.ShapeDtypeStruct((_N, 1), _f32),
            jax.ShapeDtypeStruct((_N_NEW, 1), _f32),
        ],
        compiler_params=pltpu.CompilerParams(
            dimension_semantics=("arbitrary",)),
    )(node_adj, edge_adj)


def _it_body(x0_ref, x1_ref, x2_ref, *rest):
    param_refs = rest[:18]
    dtail_ref = rest[18]
    hin_ref = rest[19]
    out_ref = rest[20]
    xs = (x0_ref, x1_ref, x2_ref)
    out_ref[0:_N_NEW, :] = hin_ref[:, :]
    for i in range(3):
        w1t, b1, gamma, beta, w2t, b2 = param_refs[6 * i:6 * i + 6]
        h1 = jnp.dot(xs[i][:, :], w1t[:, :], preferred_element_type=_f32) + b1[:, :]
        mu = jnp.mean(h1, axis=0, keepdims=True)
        var = jnp.mean((h1 - mu) ** 2, axis=0, keepdims=True)
        hn = (h1 - mu) / jnp.sqrt(var + 1e-5) * gamma[:, :] + beta[:, :]
        hr = jnp.maximum(hn, 0.0)
        h2 = jnp.dot(hr, w2t[:, :], preferred_element_type=_f32) + b2[:, :]
        out_ref[_N_NEW:_N, _NH * i:_NH * (i + 1)] = dtail_ref[:, :] * h2


def _input_transform(x, it_params, d_tail, h_in):
    x0 = x[:, 0:8]
    x1 = jnp.pad(x[:, 8:10], ((0, 0), (0, 6)))
    x2 = x[:, 10:138]
    args = [x0, x1, x2]
    for i in range(3):
        p = it_params[i]
        w1 = p["W1"]
        if w1.shape[1] == 2:
            w1 = jnp.pad(w1, ((0, 0), (0, 6)))
        args.append(w1.T)
        args.append(p["b1"].reshape(1, _NH))
        args.append(p["gamma"].reshape(1, _NH))
        args.append(p["beta"].reshape(1, _NH))
        args.append(p["W2"].T)
        args.append(p["b2"].reshape(1, _NH))
    args.append(d_tail)
    args.append(h_in)
    return pl.pallas_call(
        _it_body,
        out_shape=jax.ShapeDtypeStruct((_N, _D3), _f32),
    )(*args)


def _mega_body(n0_ref, n1_ref, e0_ref, e1_ref, hf_ref, dn_ref, de_ref,
               wzt_ref, uzt_ref, wrt_ref, urt_ref, wnt_ref, unt_ref,
               bz_ref, br_ref, bn_ref, wno_ref, weo_ref, bno_ref, beo_ref,
               ho_ref, z0_ref, z1_ref, z2_ref, y_ref, sig_ref):
    i = pl.program_id(0)
    row0 = i * _ROW_B
    a = jnp.concatenate([n0_ref[:, :] + e0_ref[:, :],
                         n1_ref[:, :] + e1_ref[:, :]], axis=0)
    m = jnp.dot(a, hf_ref[:, :], preferred_element_type=_f32)
    h = hf_ref[pl.ds(row0, _ROW_B), :]
    dn = dn_ref[pl.ds(row0, _ROW_B), :]
    de = de_ref[pl.ds(row0, _ROW_B), :]
    z = jax.nn.sigmoid(jnp.dot(m, wzt_ref[:, :], preferred_element_type=_f32)
                       + jnp.dot(h, uzt_ref[:, :], preferred_element_type=_f32)
                       + bz_ref[:, :])
    r = jax.nn.sigmoid(jnp.dot(m, wrt_ref[:, :], preferred_element_type=_f32)
                       + jnp.dot(h, urt_ref[:, :], preferred_element_type=_f32)
                       + br_ref[:, :])
    n = jnp.tanh(jnp.dot(m, wnt_ref[:, :], preferred_element_type=_f32)
                 + jnp.dot(r * h, unt_ref[:, :], preferred_element_type=_f32)
                 + bn_ref[:, :])
    ho = (1.0 - z) * h + z * n
    yv = (dn * (jnp.dot(ho, wno_ref[:, :], preferred_element_type=_f32)
                + bno_ref[:, :])
          + de * (jnp.dot(ho, weo_ref[:, :], preferred_element_type=_f32)
                  + beo_ref[:, :]))
    ho_ref[:, :] = ho
    z0_ref[:, :] = z[:, 0:_NH]
    z1_ref[:, :] = z[:, _NH:2 * _NH]
    z2_ref[:, :] = z[:, 2 * _NH:3 * _NH]
    y_ref[:, :] = yv
    sig_ref[:, :] = jax.nn.sigmoid(yv)


def _block_diag_t(mats):
    out = jnp.zeros((_D3, _D3), _f32)
    for i, m in enumerate(mats):
        out = out.at[_NH * i:_NH * (i + 1), _NH * i:_NH * (i + 1)].set(m.T)
    return out


def _mega(node_adj, edge_adj, h_full, dn, de, gru_params, out_node, out_edge):
    nblk = _N // _ROW_B
    wargs = []
    for name in ("Wz", "Uz", "Wr", "Ur", "Wn", "Un"):
        wargs.append(_block_diag_t([gru_params[i][name] for i in range(3)]))
    for name in ("bz", "br", "bn"):
        wargs.append(jnp.concatenate(
            [gru_params[i][name] for i in range(3)]).reshape(1, _D3))
    wargs.append(out_node["W"].T)
    wargs.append(out_edge["W"].T)
    wargs.append(out_node["b"].reshape(1, 1))
    wargs.append(out_edge["b"].reshape(1, 1))

    sub0 = pl.BlockSpec((_ROW_B // 2, _N), lambda i: (2 * i, 0))
    sub1 = pl.BlockSpec((_ROW_B // 2, _N), lambda i: (2 * i + 1, 0))
    full_spec = lambda shape: pl.BlockSpec(shape, lambda i: (0, 0))
    in_specs = [
        sub0,
        sub1,
        sub0,
        sub1,
        full_spec((_N, _D3)),
        full_spec((_N, 1)),
        full_spec((_N, 1)),
    ]
    in_specs += [full_spec((_D3, _D3))] * 6
    in_specs += [full_spec((1, _D3))] * 3
    in_specs += [full_spec((_D3, 1))] * 2
    in_specs += [full_spec((1, 1))] * 2
    out_specs = [
        pl.BlockSpec((_ROW_B, _D3), lambda i: (i, 0)),
        pl.BlockSpec((_ROW_B, _NH), lambda i: (i, 0)),
        pl.BlockSpec((_ROW_B, _NH), lambda i: (i, 0)),
        pl.BlockSpec((_ROW_B, _NH), lambda i: (i, 0)),
        pl.BlockSpec((_ROW_B, 1), lambda i: (i, 0)),
        pl.BlockSpec((_ROW_B, 1), lambda i: (i, 0)),
    ]
    out_shape = [
        jax.ShapeDtypeStruct((_N, _D3), _f32),
        jax.ShapeDtypeStruct((_N, _NH), _f32),
        jax.ShapeDtypeStruct((_N, _NH), _f32),
        jax.ShapeDtypeStruct((_N, _NH), _f32),
        jax.ShapeDtypeStruct((_N, 1), _f32),
        jax.ShapeDtypeStruct((_N, 1), _f32),
    ]
    return pl.pallas_call(
        _mega_body,
        grid=(nblk,),
        in_specs=in_specs,
        out_specs=out_specs,
        out_shape=out_shape,
        compiler_params=pltpu.CompilerParams(
            dimension_semantics=("parallel",),
            vmem_limit_bytes=63 * 1024 * 1024),
    )(node_adj, node_adj, edge_adj, edge_adj, h_full, dn, de, *wargs)


def kernel(x, h_in, node_adj, edge_adj, params):
    dn, de, d_tail = _extract_diags(node_adj, edge_adj)
    h_full = _input_transform(x, params["it"], d_tail, h_in)
    ho, z0, z1, z2, y, sig = _mega(node_adj, edge_adj, h_full, dn, de,
                                   params["gru"], params["out_node"],
                                   params["out_edge"])
    return sig, y, ho, (z0, z1, z2)
